# baseline (speedup 1.0000x reference)
import functools

import jax
import jax.numpy as jnp
import numpy as np
from jax import lax
from jax.experimental import pallas as pl
from jax.experimental.pallas import tpu as pltpu
from jax.experimental.pallas import tpu_sc as plsc

VOCAB = 1000000
EMBED = 128
NUM_DENSE = 13
N_SPARSE = 26
BATCH = 4096
NFEAT = 1 + N_SPARSE
NFPAD = 32

B_BLK = 256
NSPLIT = 1


def _triu_perm_scale():
    remap = lambda i: N_SPARSE if i == 0 else i - 1
    perm = np.zeros((NFPAD, NFPAD), np.int32)
    scale = np.zeros((NFPAD, NFPAD), np.float32)
    p = 0
    for i in range(NFEAT):
        for j in range(i, NFEAT):
            ni, nj = remap(i), remap(j)
            perm[ni, nj] = p
            perm[nj, ni] = p
            s = 1.0 if i == j else 0.5
            scale[ni, nj] = s
            scale[nj, ni] = s
            p += 1
    return perm.reshape(-1), scale.reshape(-1, 1)


_PERM, _SCALE = _triu_perm_scale()


def _sc_gather(table, idx, ch):
    info = plsc.get_sparse_core_info()
    nc, ns = info.num_cores, info.num_subcores
    nw = nc * ns
    b = idx.shape[0]
    b_per_w = b // nw
    n_ch = b_per_w // ch
    mesh = plsc.VectorSubcoreMesh(core_axis_name="c", subcore_axis_name="s")

    @functools.partial(
        pl.kernel,
        mesh=mesh,
        out_type=jax.ShapeDtypeStruct((b, EMBED), jnp.float32),
        scratch_types=[
            pltpu.VMEM((b_per_w,), jnp.int32),
            pltpu.VMEM((ch, EMBED), jnp.float32),
            pltpu.VMEM((ch, EMBED), jnp.float32),
            pltpu.VMEM((ch, EMBED), jnp.float32),
            pltpu.SemaphoreType.DMA,
            pltpu.SemaphoreType.DMA,
            pltpu.SemaphoreType.DMA,
            pltpu.SemaphoreType.DMA,
            pltpu.SemaphoreType.DMA,
            pltpu.SemaphoreType.DMA,
        ],
    )
    def k(table_hbm, idx_hbm, out_hbm, idx_v, buf0, buf1, buf2,
          rd0, rd1, rd2, wr0, wr1, wr2):
        wid = lax.axis_index("s") * nc + lax.axis_index("c")
        base = wid * b_per_w
        pltpu.sync_copy(idx_hbm.at[pl.ds(base, b_per_w)], idx_v)
        bufs = (buf0, buf1, buf2)
        rds = (rd0, rd1, rd2)
        wrs = (wr0, wr1, wr2)
        nb = 3
        gathers = [None] * n_ch
        writes = [None] * n_ch
        for c in range(n_ch):
            if c >= nb:
                writes[c - nb].wait()
            gathers[c] = pltpu.async_copy(
                table_hbm.at[idx_v.at[pl.ds(c * ch, ch)]],
                bufs[c % nb], rds[c % nb])
            if c >= 1:
                gathers[c - 1].wait()
                writes[c - 1] = pltpu.async_copy(
                    bufs[(c - 1) % nb], out_hbm.at[pl.ds(base + (c - 1) * ch, ch)],
                    wrs[(c - 1) % nb])
        gathers[n_ch - 1].wait()
        writes[n_ch - 1] = pltpu.async_copy(
            bufs[(n_ch - 1) % nb], out_hbm.at[pl.ds(base + (n_ch - 1) * ch, ch)],
            wrs[(n_ch - 1) % nb])
        for c in range(max(0, n_ch - nb), n_ch):
            writes[c].wait()

    return k(table, idx)


def _tc_body(dense_ref, embed_ref, wb0, bb0, wb1, bb1, wb2, bb2,
             w0a, m_ref, bt0, wt1, bt1, wt2, bt2, wt3, bt3, wt4, bt4,
             out_ref):
    f32 = jnp.float32
    bot = dense_ref[...]
    for w, b in ((wb0, bb0), (wb1, bb1), (wb2, bb2)):
        bot = jnp.maximum(
            jnp.dot(bot, w[...], preferred_element_type=f32) + b[...], 0.0)
    e_pad = embed_ref[...].reshape(B_BLK, NFPAD, EMBED)
    fidx = lax.broadcasted_iota(jnp.int32, (B_BLK, NFPAD, EMBED), 1)
    g = jnp.where(fidx == N_SPARSE, bot.reshape(B_BLK, 1, EMBED), e_pad)
    xact = jax.lax.dot_general(
        g, g, (((2,), (2,)), ((0,), (0,))),
        preferred_element_type=f32)
    xf = xact.reshape(B_BLK, NFPAD * NFPAD)
    t = (jnp.dot(xf, m_ref[...], preferred_element_type=f32)
         + jnp.dot(bot, w0a[...], preferred_element_type=f32) + bt0[...])
    t = jnp.maximum(t, 0.0)
    for i, (w, b) in enumerate(((wt1, bt1), (wt2, bt2), (wt3, bt3), (wt4, bt4))):
        t = jnp.dot(t, w[...], preferred_element_type=f32) + b[...]
        if i < 3:
            t = jnp.maximum(t, 0.0)
    out_ref[...] = t


def _tc_forward(dense, embed, wb0, bb0, wb1, bb1, wb2, bb2,
                w0a, m, bt0, wt1, bt1, wt2, bt2, wt3, bt3, wt4, bt4):
    nbatch = dense.shape[0]
    grid = nbatch // B_BLK
    inv = lambda shape: pl.BlockSpec(shape, lambda i: (0,) * len(shape))
    in_specs = [
        pl.BlockSpec((B_BLK, NUM_DENSE), lambda i: (i, 0)),
        pl.BlockSpec((B_BLK * NFPAD, EMBED), lambda i: (i, 0)),
        inv(wb0.shape), inv(bb0.shape), inv(wb1.shape), inv(bb1.shape),
        inv(wb2.shape), inv(bb2.shape),
        inv(w0a.shape), inv(m.shape), inv(bt0.shape),
        inv(wt1.shape), inv(bt1.shape), inv(wt2.shape), inv(bt2.shape),
        inv(wt3.shape), inv(bt3.shape), inv(wt4.shape), inv(bt4.shape),
    ]
    return pl.pallas_call(
        _tc_body,
        grid=(grid,),
        in_specs=in_specs,
        out_specs=pl.BlockSpec((B_BLK, 1), lambda i: (i, 0)),
        out_shape=jax.ShapeDtypeStruct((nbatch, 1), jnp.float32),
        compiler_params=pltpu.CompilerParams(
            dimension_semantics=("arbitrary",)),
    )(dense, embed, wb0, bb0, wb1, bb1, wb2, bb2,
      w0a, m, bt0, wt1, bt1, wt2, bt2, wt3, bt3, wt4, bt4)


def kernel(x, Wb0, bb0, Wb1, bb1, Wb2, bb2, embedding_table,
           Wt0, bt0, Wt1, bt1, Wt2, bt2, Wt3, bt3, Wt4, bt4, train=False):
    del train
    dense = x[:, :NUM_DENSE]
    cat = x[:, NUM_DENSE:].astype(jnp.int32)
    idx3 = cat % VOCAB
    idx_pad = jnp.concatenate(
        [idx3, idx3[:, :NFPAD - N_SPARSE]], axis=1).reshape(-1)

    w0a = Wt0[:EMBED]
    m = Wt0[EMBED:][jnp.asarray(_PERM)] * jnp.asarray(_SCALE)

    def r2(b):
        return b.reshape(1, -1)

    bh = BATCH // NSPLIT
    ih = bh * NFPAD
    embeds = [_sc_gather(embedding_table, idx_pad[k * ih:(k + 1) * ih], 256)
              for k in range(NSPLIT)]
    outs = [
        _tc_forward(dense[k * bh:(k + 1) * bh], embeds[k],
                    Wb0, r2(bb0), Wb1, r2(bb1), Wb2, r2(bb2),
                    w0a, m, r2(bt0), Wt1, r2(bt1), Wt2, r2(bt2),
                    Wt3, r2(bt3), Wt4, r2(bt4))
        for k in range(NSPLIT)
    ]
    return jnp.concatenate(outs, axis=0)


# speedup vs baseline: 1.0570x; 1.0570x over previous
import functools

import jax
import jax.numpy as jnp
import numpy as np
from jax import lax
from jax.experimental import pallas as pl
from jax.experimental.pallas import tpu as pltpu
from jax.experimental.pallas import tpu_sc as plsc

VOCAB = 1000000
EMBED = 128
NUM_DENSE = 13
N_SPARSE = 26
BATCH = 4096
NFEAT = 1 + N_SPARSE
NFPAD = 32

B_BLK = 256
NSPLIT = 2


NPAIR_TRI = (NFEAT * (NFEAT + 1)) // 2


def _pair_projection():
    remap = lambda i: N_SPARSE if i == 0 else i - 1
    proj = np.zeros((NFPAD, NFPAD, NPAIR_TRI), np.float32)
    p = 0
    for i in range(NFEAT):
        for j in range(i, NFEAT):
            ni, nj = remap(i), remap(j)
            s = 1.0 if i == j else 0.5
            proj[ni, nj, p] = s
            proj[nj, ni, p] = s
            p += 1
    return proj.reshape(NFPAD * NFPAD, NPAIR_TRI)


_PROJ = _pair_projection()


def _sc_gather(table, idx, ch):
    info = plsc.get_sparse_core_info()
    nc, ns = info.num_cores, info.num_subcores
    nw = nc * ns
    b = idx.shape[0]
    b_per_w = b // nw
    n_ch = b_per_w // ch
    mesh = plsc.VectorSubcoreMesh(core_axis_name="c", subcore_axis_name="s")

    @functools.partial(
        pl.kernel,
        mesh=mesh,
        out_type=jax.ShapeDtypeStruct((b, EMBED), jnp.float32),
        scratch_types=[
            pltpu.VMEM((b_per_w,), jnp.int32),
            pltpu.VMEM((ch, EMBED), jnp.float32),
            pltpu.VMEM((ch, EMBED), jnp.float32),
            pltpu.VMEM((ch, EMBED), jnp.float32),
            pltpu.SemaphoreType.DMA,
            pltpu.SemaphoreType.DMA,
            pltpu.SemaphoreType.DMA,
            pltpu.SemaphoreType.DMA,
            pltpu.SemaphoreType.DMA,
            pltpu.SemaphoreType.DMA,
        ],
    )
    def k(table_hbm, idx_hbm, out_hbm, idx_v, buf0, buf1, buf2,
          rd0, rd1, rd2, wr0, wr1, wr2):
        wid = lax.axis_index("s") * nc + lax.axis_index("c")
        base = wid * b_per_w
        pltpu.sync_copy(idx_hbm.at[pl.ds(base, b_per_w)], idx_v)
        bufs = (buf0, buf1, buf2)
        rds = (rd0, rd1, rd2)
        wrs = (wr0, wr1, wr2)
        nb = 3
        gathers = [None] * n_ch
        writes = [None] * n_ch
        for c in range(n_ch):
            if c >= nb:
                writes[c - nb].wait()
            gathers[c] = pltpu.async_copy(
                table_hbm.at[idx_v.at[pl.ds(c * ch, ch)]],
                bufs[c % nb], rds[c % nb])
            if c >= 1:
                gathers[c - 1].wait()
                writes[c - 1] = pltpu.async_copy(
                    bufs[(c - 1) % nb], out_hbm.at[pl.ds(base + (c - 1) * ch, ch)],
                    wrs[(c - 1) % nb])
        gathers[n_ch - 1].wait()
        writes[n_ch - 1] = pltpu.async_copy(
            bufs[(n_ch - 1) % nb], out_hbm.at[pl.ds(base + (n_ch - 1) * ch, ch)],
            wrs[(n_ch - 1) % nb])
        for c in range(max(0, n_ch - nb), n_ch):
            writes[c].wait()

    return k(table, idx)


def _tc_body(dense_ref, embed_ref, wb0, bb0, wb1, bb1, wb2, bb2,
             w0a, proj_ref, w0b, bt0, wt1, bt1, wt2, bt2, wt3, bt3, wt4, bt4,
             out_ref):
    f32 = jnp.float32
    bot = dense_ref[...]
    for w, b in ((wb0, bb0), (wb1, bb1), (wb2, bb2)):
        bot = jnp.maximum(
            jnp.dot(bot, w[...], preferred_element_type=f32) + b[...], 0.0)
    e_pad = embed_ref[...].reshape(B_BLK, NFPAD, EMBED)
    fidx = lax.broadcasted_iota(jnp.int32, (B_BLK, NFPAD, EMBED), 1)
    g = jnp.where(fidx == N_SPARSE, bot.reshape(B_BLK, 1, EMBED), e_pad)
    xact = jax.lax.dot_general(
        g, g, (((2,), (2,)), ((0,), (0,))),
        preferred_element_type=f32)
    xf = xact.reshape(B_BLK, NFPAD * NFPAD)
    di = jnp.dot(xf, proj_ref[...], preferred_element_type=f32)
    t = (jnp.dot(di, w0b[...], preferred_element_type=f32)
         + jnp.dot(bot, w0a[...], preferred_element_type=f32) + bt0[...])
    t = jnp.maximum(t, 0.0)
    for i, (w, b) in enumerate(((wt1, bt1), (wt2, bt2), (wt3, bt3), (wt4, bt4))):
        t = jnp.dot(t, w[...], preferred_element_type=f32) + b[...]
        if i < 3:
            t = jnp.maximum(t, 0.0)
    out_ref[...] = t


def _tc_forward(dense, embed, wb0, bb0, wb1, bb1, wb2, bb2,
                w0a, proj, w0b, bt0, wt1, bt1, wt2, bt2, wt3, bt3, wt4, bt4):
    nbatch = dense.shape[0]
    grid = nbatch // B_BLK
    inv = lambda shape: pl.BlockSpec(shape, lambda i: (0,) * len(shape))
    in_specs = [
        pl.BlockSpec((B_BLK, NUM_DENSE), lambda i: (i, 0)),
        pl.BlockSpec((B_BLK * NFPAD, EMBED), lambda i: (i, 0)),
        inv(wb0.shape), inv(bb0.shape), inv(wb1.shape), inv(bb1.shape),
        inv(wb2.shape), inv(bb2.shape),
        inv(w0a.shape), inv(proj.shape), inv(w0b.shape), inv(bt0.shape),
        inv(wt1.shape), inv(bt1.shape), inv(wt2.shape), inv(bt2.shape),
        inv(wt3.shape), inv(bt3.shape), inv(wt4.shape), inv(bt4.shape),
    ]
    return pl.pallas_call(
        _tc_body,
        grid=(grid,),
        in_specs=in_specs,
        out_specs=pl.BlockSpec((B_BLK, 1), lambda i: (i, 0)),
        out_shape=jax.ShapeDtypeStruct((nbatch, 1), jnp.float32),
        compiler_params=pltpu.CompilerParams(
            dimension_semantics=("arbitrary",)),
    )(dense, embed, wb0, bb0, wb1, bb1, wb2, bb2,
      w0a, proj, w0b, bt0, wt1, bt1, wt2, bt2, wt3, bt3, wt4, bt4)


def kernel(x, Wb0, bb0, Wb1, bb1, Wb2, bb2, embedding_table,
           Wt0, bt0, Wt1, bt1, Wt2, bt2, Wt3, bt3, Wt4, bt4, train=False):
    del train
    dense = x[:, :NUM_DENSE]
    cat = x[:, NUM_DENSE:].astype(jnp.int32)
    idx3 = cat % VOCAB
    idx_pad = jnp.concatenate(
        [idx3, idx3[:, :NFPAD - N_SPARSE]], axis=1).reshape(-1)

    w0a = Wt0[:EMBED]
    w0b = Wt0[EMBED:]
    proj = jnp.asarray(_PROJ)

    def r2(b):
        return b.reshape(1, -1)

    bh = BATCH // NSPLIT
    ih = bh * NFPAD
    embeds = [_sc_gather(embedding_table, idx_pad[k * ih:(k + 1) * ih], 256)
              for k in range(NSPLIT)]
    outs = [
        _tc_forward(dense[k * bh:(k + 1) * bh], embeds[k],
                    Wb0, r2(bb0), Wb1, r2(bb1), Wb2, r2(bb2),
                    w0a, proj, w0b, r2(bt0), Wt1, r2(bt1), Wt2, r2(bt2),
                    Wt3, r2(bt3), Wt4, r2(bt4))
        for k in range(NSPLIT)
    ]
    return jnp.concatenate(outs, axis=0)
